# baseline (speedup 1.0000x reference)
import jax
import jax.numpy as jnp
from jax.experimental import pallas as pl
from jax.experimental.pallas import tpu as pltpu

BLOCK_COLS = 16384


def _copy_body(u_in, i_in, u_out, i_out):
    u_out[...] = u_in[...]
    i_out[...] = i_in[...]


def kernel(user_emb, item_emb):
    ut = user_emb.T
    it = item_emb.T
    d, n = ut.shape
    grid = (pl.cdiv(n, BLOCK_COLS),)
    spec = pl.BlockSpec((d, BLOCK_COLS), lambda g: (0, g))
    out_ut, out_it = pl.pallas_call(
        _copy_body,
        grid=grid,
        out_shape=(
            jax.ShapeDtypeStruct(ut.shape, ut.dtype),
            jax.ShapeDtypeStruct(it.shape, it.dtype),
        ),
        in_specs=[spec, spec],
        out_specs=[spec, spec],
    )(ut, it)
    return out_ut.T, out_it.T


# speedup vs baseline: 1.0233x; 1.0233x over previous
import jax
import jax.numpy as jnp
from jax.experimental import pallas as pl
from jax.experimental.pallas import tpu as pltpu

BLOCK_COLS = 49152


def _copy_body(u_in, i_in, u_out, i_out):
    u_out[...] = u_in[...]
    i_out[...] = i_in[...]


def kernel(user_emb, item_emb):
    ut = user_emb.T
    it = item_emb.T
    d, n = ut.shape
    grid = (pl.cdiv(n, BLOCK_COLS),)
    spec = pl.BlockSpec((d, BLOCK_COLS), lambda g: (0, g))
    out_ut, out_it = pl.pallas_call(
        _copy_body,
        grid=grid,
        out_shape=(
            jax.ShapeDtypeStruct(ut.shape, ut.dtype),
            jax.ShapeDtypeStruct(it.shape, it.dtype),
        ),
        in_specs=[spec, spec],
        out_specs=[spec, spec],
    )(ut, it)
    return out_ut.T, out_it.T


# speedup vs baseline: 1.0285x; 1.0051x over previous
import jax
import jax.numpy as jnp
from jax.experimental import pallas as pl
from jax.experimental.pallas import tpu as pltpu

BLOCK_COLS = 114688


def _copy_body(src, dst):
    dst[...] = src[...]


def _tc_copy(x):
    d, n = x.shape
    spec = pl.BlockSpec((d, BLOCK_COLS), lambda g: (0, g))
    return pl.pallas_call(
        _copy_body,
        grid=(pl.cdiv(n, BLOCK_COLS),),
        out_shape=jax.ShapeDtypeStruct(x.shape, x.dtype),
        in_specs=[spec],
        out_specs=spec,
    )(x)


def kernel(user_emb, item_emb):
    ut = user_emb.T
    it = item_emb.T
    return _tc_copy(ut).T, _tc_copy(it).T


# speedup vs baseline: 1.0312x; 1.0025x over previous
import jax
from jax.experimental import pallas as pl

BLOCK_COLS = 118784


def _copy_body(src, dst):
    dst[...] = src[...]


def _tc_copy(x):
    d, n = x.shape
    spec = pl.BlockSpec((d, BLOCK_COLS), lambda g: (0, g))
    return pl.pallas_call(
        _copy_body,
        grid=(pl.cdiv(n, BLOCK_COLS),),
        out_shape=jax.ShapeDtypeStruct(x.shape, x.dtype),
        in_specs=[spec],
        out_specs=spec,
    )(x)


def kernel(user_emb, item_emb):
    ut = user_emb.T
    it = item_emb.T
    return _tc_copy(ut).T, _tc_copy(it).T
